# baseline (speedup 1.0000x reference)
import functools

import jax
import jax.numpy as jnp
from jax import lax
from jax.experimental import pallas as pl
from jax.experimental.pallas import tpu as pltpu
from jax.experimental.pallas import tpu_sc as plsc

GAMMA = 12.0
HIDDEN = 128
BATCH = 16384
NUM_WORKERS = 32
SAMPLES_PER_W = BATCH // NUM_WORKERS
CHUNK = 128
NCHUNK = SAMPLES_PER_W // CHUNK
GRP = 16

_mesh = plsc.VectorSubcoreMesh(core_axis_name="c", subcore_axis_name="s")


@functools.partial(
    pl.kernel,
    mesh=_mesh,
    out_type=jax.ShapeDtypeStruct((BATCH,), jnp.float32),
    compiler_params=pltpu.CompilerParams(needs_layout_passes=False),
    scratch_types=[
        pltpu.VMEM((SAMPLES_PER_W,), jnp.int32),
        pltpu.VMEM((SAMPLES_PER_W,), jnp.int32),
        pltpu.VMEM((SAMPLES_PER_W,), jnp.int32),
        pltpu.VMEM((2, CHUNK, HIDDEN), jnp.float32),
        pltpu.VMEM((2, CHUNK, HIDDEN), jnp.float32),
        pltpu.VMEM((2, CHUNK, HIDDEN), jnp.float32),
        pltpu.VMEM((SAMPLES_PER_W,), jnp.float32),
        pltpu.SemaphoreType.DMA,
        pltpu.SemaphoreType.DMA,
        pltpu.SemaphoreType.DMA,
    ],
)
def _score_kernel(ent_hbm, rel_hbm, hidx_hbm, ridx_hbm, tidx_hbm, out_hbm,
                  ih, ir, it, hv, rv, tv, outv, sem0, sem1, semi):
    wid = lax.axis_index("s") * 2 + lax.axis_index("c")
    base = wid * SAMPLES_PER_W

    ci_h = pltpu.async_copy(hidx_hbm.at[pl.ds(base, SAMPLES_PER_W)], ih, semi)
    ci_r = pltpu.async_copy(ridx_hbm.at[pl.ds(base, SAMPLES_PER_W)], ir, semi)
    ci_t = pltpu.async_copy(tidx_hbm.at[pl.ds(base, SAMPLES_PER_W)], it, semi)
    ci_h.wait()
    ci_r.wait()
    ci_t.wait()

    sems = (sem0, sem1)

    def fire(c):
        p = c % 2
        sl = pl.ds(c * CHUNK, CHUNK)
        return (
            pltpu.async_copy(ent_hbm.at[ih.at[sl]], hv.at[p], sems[p]),
            pltpu.async_copy(rel_hbm.at[ir.at[sl]], rv.at[p], sems[p]),
            pltpu.async_copy(ent_hbm.at[it.at[sl]], tv.at[p], sems[p]),
        )

    last_lane = lax.iota(jnp.int32, 16) == 15
    inflight = fire(0)
    for c in range(NCHUNK):
        nxt = fire(c + 1) if c + 1 < NCHUNK else None
        for cp in inflight:
            cp.wait()
        inflight = nxt
        p = c % 2

        @plsc.parallel_loop(0, CHUNK, unroll=4)
        def s_body(s):
            acc0 = jnp.zeros((16,), jnp.float32)
            acc1 = jnp.zeros((16,), jnp.float32)
            for j in range(HIDDEN // 32):
                d0 = pl.ds(j * 32, 16)
                d1 = pl.ds(j * 32 + 16, 16)
                acc0 = acc0 + jnp.abs(hv[p, s, d0] + rv[p, s, d0]
                                      - tv[p, s, d0])
                acc1 = acc1 + jnp.abs(hv[p, s, d1] + rv[p, s, d1]
                                      - tv[p, s, d1])
            score = GAMMA - jnp.cumsum(acc0 + acc1)
            pos = jnp.full((16,), c * CHUNK + s, jnp.int32)
            plsc.store_scatter(outv, [pos], score, mask=last_lane)

    pltpu.sync_copy(outv, out_hbm.at[pl.ds(base, SAMPLES_PER_W)])


def kernel(entity_embedding, relation_embedding, sample):
    h = sample[:, 0].astype(jnp.int32)
    r = sample[:, 1].astype(jnp.int32)
    t = sample[:, 2].astype(jnp.int32)
    out = _score_kernel(entity_embedding, relation_embedding, h, r, t)
    return out.reshape(BATCH, 1)


# speedup vs baseline: 1.0224x; 1.0224x over previous
import functools

import jax
import jax.numpy as jnp
from jax import lax
from jax.experimental import pallas as pl
from jax.experimental.pallas import tpu as pltpu
from jax.experimental.pallas import tpu_sc as plsc

GAMMA = 12.0
HIDDEN = 128
BATCH = 16384
NUM_WORKERS = 32
SAMPLES_PER_W = BATCH // NUM_WORKERS
CHUNK = 128
NCHUNK = SAMPLES_PER_W // CHUNK
GRP = 16

_mesh = plsc.VectorSubcoreMesh(core_axis_name="c", subcore_axis_name="s")


@functools.partial(
    pl.kernel,
    mesh=_mesh,
    out_type=jax.ShapeDtypeStruct((BATCH,), jnp.float32),
    compiler_params=pltpu.CompilerParams(needs_layout_passes=False),
    scratch_types=[
        pltpu.VMEM((SAMPLES_PER_W,), jnp.int32),
        pltpu.VMEM((SAMPLES_PER_W,), jnp.int32),
        pltpu.VMEM((SAMPLES_PER_W,), jnp.int32),
        pltpu.VMEM((2, CHUNK, HIDDEN), jnp.float32),
        pltpu.VMEM((2, CHUNK, HIDDEN), jnp.float32),
        pltpu.VMEM((2, CHUNK, HIDDEN), jnp.float32),
        pltpu.VMEM((SAMPLES_PER_W,), jnp.float32),
        pltpu.SemaphoreType.DMA,
        pltpu.SemaphoreType.DMA,
        pltpu.SemaphoreType.DMA,
    ],
)
def _score_kernel(ent_hbm, rel_hbm, hidx_hbm, ridx_hbm, tidx_hbm, out_hbm,
                  ih, ir, it, hv, rv, tv, outv, sem0, sem1, semi):
    wid = lax.axis_index("s") * 2 + lax.axis_index("c")
    base = wid * SAMPLES_PER_W

    ci_h = pltpu.async_copy(hidx_hbm.at[pl.ds(base, SAMPLES_PER_W)], ih, semi)
    ci_r = pltpu.async_copy(ridx_hbm.at[pl.ds(base, SAMPLES_PER_W)], ir, semi)
    ci_t = pltpu.async_copy(tidx_hbm.at[pl.ds(base, SAMPLES_PER_W)], it, semi)
    ci_h.wait()
    ci_r.wait()
    ci_t.wait()

    sems = (sem0, sem1)

    def fire(c):
        p = c % 2
        sl = pl.ds(c * CHUNK, CHUNK)
        return (
            pltpu.async_copy(ent_hbm.at[ih.at[sl]], hv.at[p], sems[p]),
            pltpu.async_copy(rel_hbm.at[ir.at[sl]], rv.at[p], sems[p]),
            pltpu.async_copy(ent_hbm.at[it.at[sl]], tv.at[p], sems[p]),
        )

    last_lane = lax.iota(jnp.int32, 16) == 15
    inflight = fire(0)
    for c in range(NCHUNK):
        nxt = fire(c + 1) if c + 1 < NCHUNK else None
        for cp in inflight:
            cp.wait()
        inflight = nxt
        p = c % 2

        @plsc.parallel_loop(0, CHUNK, unroll=2)
        def s_body(s):
            acc0 = jnp.zeros((16,), jnp.float32)
            acc1 = jnp.zeros((16,), jnp.float32)
            for j in range(HIDDEN // 32):
                d0 = pl.ds(j * 32, 16)
                d1 = pl.ds(j * 32 + 16, 16)
                acc0 = acc0 + jnp.abs(hv[p, s, d0] + rv[p, s, d0]
                                      - tv[p, s, d0])
                acc1 = acc1 + jnp.abs(hv[p, s, d1] + rv[p, s, d1]
                                      - tv[p, s, d1])
            score = GAMMA - jnp.cumsum(acc0 + acc1)
            pos = jnp.full((16,), c * CHUNK + s, jnp.int32)
            plsc.store_scatter(outv, [pos], score, mask=last_lane)

    pltpu.sync_copy(outv, out_hbm.at[pl.ds(base, SAMPLES_PER_W)])


def kernel(entity_embedding, relation_embedding, sample):
    h = sample[:, 0].astype(jnp.int32)
    r = sample[:, 1].astype(jnp.int32)
    t = sample[:, 2].astype(jnp.int32)
    out = _score_kernel(entity_embedding, relation_embedding, h, r, t)
    return out.reshape(BATCH, 1)
